# baseline (speedup 1.0000x reference)
import functools

import jax
import jax.numpy as jnp
from jax import lax
from jax.experimental import pallas as pl
from jax.experimental.pallas import tpu as pltpu
from jax.experimental.pallas import tpu_sc as plsc

NN = 50000
NPAD = 50176
EE = 800000
EPAD = 819200
ROWS = 1024
GRID = NPAD // ROWS
NC, NS = 2, 16
PT = NC * NS
EPT = EPAD // PT
QW = NPAD // 4
QACC = QW + 16
PAD = 6912
GRE = 48
NB = 4
IDXS = 8
NGRP = PAD // GRE
NCYC = NGRP // NB
NG = 64


def _sc_partition():
    mesh = plsc.VectorSubcoreMesh(
        core_axis_name="c", subcore_axis_name="s", num_cores=NC, num_subcores=NS
    )

    @functools.partial(
        pl.kernel,
        out_type=[jax.ShapeDtypeStruct((4, PT, PAD), jnp.int32)] * 2,
        mesh=mesh,
        compiler_params=pltpu.CompilerParams(
            use_tc_tiling_on_sc=False, needs_layout_passes=False
        ),
        scratch_types=(
            [
                pltpu.VMEM((EPT,), jnp.int32),
                pltpu.VMEM((EPT,), jnp.int32),
            ]
            + [pltpu.VMEM((PAD + 16,), jnp.int32) for _ in range(8)]
        ),
    )
    def part_kernel(src_hbm, dst_hbm, qsrc_out, qdst_out, esrc, edst, *qb):
        qsb = qb[:4]
        qdb = qb[4:]
        cid = lax.axis_index("c")
        sid = lax.axis_index("s")
        wid = sid * NC + cid
        base = wid * EPT

        pltpu.sync_copy(src_hbm.at[pl.ds(base, EPT)], esrc)
        pltpu.sync_copy(dst_hbm.at[pl.ds(base, EPT)], edst)

        ones = jnp.ones((16,), jnp.int32)
        zeros16 = jnp.zeros((16,), jnp.int32)

        def scan_step(t, offs):
            s16 = esrc[pl.ds(t * 16, 16)]
            d16 = edst[pl.ds(t * 16, 16)]
            qv = (
                jnp.where(d16 >= QW, ones, zeros16)
                + jnp.where(d16 >= 2 * QW, ones, zeros16)
                + jnp.where(d16 >= 3 * QW, ones, zeros16)
            )
            dl = d16 - qv * QW
            new_offs = []
            for q in range(4):
                m = qv == q
                c = plsc.cumsum(jnp.where(m, ones, zeros16))
                pos = jnp.where(
                    m, jnp.minimum(offs[q] + c - 1, PAD + 15), 0
                )
                plsc.store_scatter(qsb[q], (pos,), s16, mask=m)
                plsc.store_scatter(qdb[q], (pos,), dl, mask=m)
                new_offs.append(offs[q] + c[15])
            return tuple(new_offs)

        offs = lax.fori_loop(
            0,
            EPT // 16,
            scan_step,
            (jnp.int32(0), jnp.int32(0), jnp.int32(0), jnp.int32(0)),
        )

        iota16 = lax.iota(jnp.int32, 16)
        dsrc = jnp.zeros((16,), jnp.int32)
        ddst = jnp.full((16,), QW, jnp.int32)
        for q in range(4):

            def fill_body(j, off, q=q):
                fmask = off < PAD
                fpos = jnp.minimum(off + iota16, PAD + 15)
                plsc.store_scatter(
                    qsb[q], (fpos,), dsrc, mask=jnp.full((16,), True) & fmask
                )
                plsc.store_scatter(
                    qdb[q], (fpos,), ddst, mask=jnp.full((16,), True) & fmask
                )
                return jnp.where(fmask, off + 16, off)

            lax.fori_loop(0, (PAD + 15) // 16, fill_body, offs[q])
            pltpu.sync_copy(qsb[q].at[pl.ds(0, PAD)], qsrc_out.at[q, wid])
            pltpu.sync_copy(qdb[q].at[pl.ds(0, PAD)], qdst_out.at[q, wid])

        return None

    return part_kernel


def _sc_agg(W):
    mesh = plsc.VectorSubcoreMesh(
        core_axis_name="c", subcore_axis_name="s", num_cores=NC, num_subcores=NS
    )
    zrpt = QACC // NS
    wrpt = QW // NS

    @functools.partial(
        pl.kernel,
        out_type=jax.ShapeDtypeStruct((NPAD, W), jnp.float32),
        mesh=mesh,
        compiler_params=pltpu.CompilerParams(use_tc_tiling_on_sc=False),
        scratch_types=(
            [
                pltpu.VMEM((IDXS, GRE), jnp.int32),
                pltpu.VMEM((IDXS, GRE), jnp.int32),
                pltpu.VMEM_SHARED((QACC, W), jnp.float32),
            ]
            + [pltpu.VMEM((GRE, W), jnp.float32) for _ in range(NB)]
            + [pltpu.SemaphoreType.DMA] * (2 + 2 * NB)
        ),
    )
    def agg_kernel(qsrc_hbm, qdst_hbm, zeros_hbm, h_hbm, out_ref, *rest):
        sbuf, dbuf, acc = rest[:3]
        rows = rest[3:3 + NB]
        sem_si, sem_di = rest[3 + NB:5 + NB]
        sem_g = rest[5 + NB:5 + 2 * NB]
        sem_s = rest[5 + 2 * NB:5 + 3 * NB]

        cid = lax.axis_index("c")
        sid = lax.axis_index("s")

        def fire_gather(b, islot):
            return pltpu.async_copy(
                h_hbm.at[sbuf.at[islot]], rows[b], sem_g[b]
            )

        def wait_gather(b, islot):
            pltpu.make_async_copy(
                h_hbm.at[sbuf.at[islot]], rows[b], sem_g[b]
            ).wait()

        def fire_scatter(b, islot):
            return pltpu.async_copy(
                rows[b], acc.at[dbuf.at[islot]], sem_s[b], add=True
            )

        def wait_scatter(b, islot):
            pltpu.make_async_copy(
                rows[b], acc.at[dbuf.at[islot]], sem_s[b]
            ).wait()

        for q in range(4):

            @pl.when(cid == q // 2)
            def _(q=q):
                pltpu.sync_copy(zeros_hbm, acc.at[pl.ds(sid * zrpt, zrpt)])
                plsc.subcore_barrier()

                for sl in range(2):
                    slot = sid * 2 + sl

                    def fire_idx(g, islot, slot=slot, q=q):
                        pltpu.async_copy(
                            qsrc_hbm.at[q, slot, pl.ds(g * GRE, GRE)],
                            sbuf.at[islot],
                            sem_si,
                        )
                        pltpu.async_copy(
                            qdst_hbm.at[q, slot, pl.ds(g * GRE, GRE)],
                            dbuf.at[islot],
                            sem_di,
                        )

                    def wait_idx(islot, slot=slot, q=q):
                        pltpu.make_async_copy(
                            qsrc_hbm.at[q, slot, pl.ds(0, GRE)],
                            sbuf.at[islot],
                            sem_si,
                        ).wait()
                        pltpu.make_async_copy(
                            qdst_hbm.at[q, slot, pl.ds(0, GRE)],
                            dbuf.at[islot],
                            sem_di,
                        ).wait()

                    for j in range(IDXS):
                        fire_idx(j, j)
                    for b in range(NB):
                        wait_idx(b)
                        fire_gather(b, b)

                    def cycle(i, carry):
                        for b in range(NB):
                            g = i * NB + b
                            wait_gather(b, g % IDXS)
                            fire_scatter(b, g % IDXS)
                        for b in range(NB):
                            g = i * NB + b
                            gn = g + NB
                            wait_scatter(b, g % IDXS)

                            @pl.when(gn < NGRP)
                            def _(b=b, gn=gn):
                                wait_idx(gn % IDXS)
                                fire_gather(b, gn % IDXS)

                        for b in range(NB):
                            g = i * NB + b
                            gi = g + IDXS

                            @pl.when(gi < NGRP)
                            def _(g=g, gi=gi):
                                fire_idx(gi, g % IDXS)

                        return carry

                    lax.fori_loop(0, NCYC, cycle, 0)

                plsc.subcore_barrier()

                pltpu.sync_copy(
                    acc.at[pl.ds(sid * wrpt, wrpt)],
                    out_ref.at[pl.ds(q * QW + sid * wrpt, wrpt)],
                )
                plsc.subcore_barrier()

        return None

    return agg_kernel


def _mlp_stats(Hin, h, agg, wa, ba, wb, bb):
    def body(h_ref, a_ref, wa_ref, ba_ref, wb_ref, bb_ref, t_ref, st_ref):
        i = pl.program_id(0)
        m = h_ref[...] + a_ref[...]
        z = jnp.maximum(
            jnp.dot(m, wa_ref[...], preferred_element_type=jnp.float32)
            + ba_ref[0, :],
            0.0,
        )
        t = (
            jnp.dot(z, wb_ref[...], preferred_element_type=jnp.float32)
            + bb_ref[0, :]
        )
        t_ref[...] = t

        rowid = i * ROWS + lax.broadcasted_iota(jnp.int32, (ROWS, 1), 0)
        tm = jnp.where(rowid < NN, t, 0.0)
        s = jnp.sum(tm, axis=0)
        ss = jnp.sum(tm * tm, axis=0)
        st = jnp.concatenate([s[None, :], ss[None, :]], axis=0)

        @pl.when(i == 0)
        def _():
            st_ref[...] = jnp.zeros((2, 128), jnp.float32)

        st_ref[...] += st

    return pl.pallas_call(
        body,
        grid=(GRID,),
        in_specs=[
            pl.BlockSpec((ROWS, Hin), lambda i: (i, 0)),
            pl.BlockSpec((ROWS, Hin), lambda i: (i, 0)),
            pl.BlockSpec((Hin, 128), lambda i: (0, 0)),
            pl.BlockSpec((1, 128), lambda i: (0, 0)),
            pl.BlockSpec((128, 128), lambda i: (0, 0)),
            pl.BlockSpec((1, 128), lambda i: (0, 0)),
        ],
        out_specs=[
            pl.BlockSpec((ROWS, 128), lambda i: (i, 0)),
            pl.BlockSpec((2, 128), lambda i: (0, 0)),
        ],
        out_shape=[
            jax.ShapeDtypeStruct((NPAD, 128), jnp.float32),
            jax.ShapeDtypeStruct((2, 128), jnp.float32),
        ],
    )(h, agg, wa, ba, wb, bb)


def _bn_relu(t, scale, shift):
    def body(t_ref, sc_ref, sh_ref, out_ref):
        out_ref[...] = jnp.maximum(
            t_ref[...] * sc_ref[0, :] + sh_ref[0, :], 0.0
        )

    return pl.pallas_call(
        body,
        grid=(GRID,),
        in_specs=[
            pl.BlockSpec((ROWS, 128), lambda i: (i, 0)),
            pl.BlockSpec((1, 128), lambda i: (0, 0)),
            pl.BlockSpec((1, 128), lambda i: (0, 0)),
        ],
        out_specs=pl.BlockSpec((ROWS, 128), lambda i: (i, 0)),
        out_shape=jax.ShapeDtypeStruct((NPAD, 128), jnp.float32),
    )(t, scale, shift)


def _bn_relu_pool(t, scale, shift, batch3d):
    def body(t_ref, sc_ref, sh_ref, b_ref, pool_ref):
        i = pl.program_id(0)
        h = jnp.maximum(t_ref[...] * sc_ref[0, :] + sh_ref[0, :], 0.0)
        seg = b_ref[0, 0, :]
        onehot = (
            lax.broadcasted_iota(jnp.int32, (NG, ROWS), 0) == seg[None, :]
        ).astype(jnp.float32)
        part = jnp.dot(onehot, h, preferred_element_type=jnp.float32)

        @pl.when(i == 0)
        def _():
            pool_ref[...] = jnp.zeros((NG, 128), jnp.float32)

        pool_ref[...] += part

    return pl.pallas_call(
        body,
        grid=(GRID,),
        in_specs=[
            pl.BlockSpec((ROWS, 128), lambda i: (i, 0)),
            pl.BlockSpec((1, 128), lambda i: (0, 0)),
            pl.BlockSpec((1, 128), lambda i: (0, 0)),
            pl.BlockSpec((1, 1, ROWS), lambda i: (i, 0, 0)),
        ],
        out_specs=pl.BlockSpec((NG, 128), lambda i: (0, 0)),
        out_shape=jax.ShapeDtypeStruct((NG, 128), jnp.float32),
    )(t, scale, shift, batch3d)


def _head(pooled, wc1, bc1, wc2, bc2):
    def body(p_ref, w1_ref, b1_ref, w2_ref, b2_ref, o_ref):
        z1 = jnp.maximum(
            jnp.dot(p_ref[...], w1_ref[...], preferred_element_type=jnp.float32)
            + b1_ref[0, :],
            0.0,
        )
        z = (
            jnp.dot(z1, w2_ref[...], preferred_element_type=jnp.float32)
            + b2_ref[0, :]
        )
        zmax = jnp.max(z, axis=1, keepdims=True)
        lse = zmax + jnp.log(jnp.sum(jnp.exp(z - zmax), axis=1, keepdims=True))
        o_ref[...] = z - lse

    return pl.pallas_call(
        body,
        out_shape=jax.ShapeDtypeStruct((NG, 2), jnp.float32),
    )(pooled, wc1, bc1, wc2, bc2)


def kernel(x, edge_index, batch, params):
    layers, head = params
    src = edge_index[0].astype(jnp.int32)
    dst = edge_index[1].astype(jnp.int32)

    src1d = jnp.concatenate([src, jnp.zeros((EPAD - EE,), jnp.int32)])
    dst1d = jnp.concatenate([dst, jnp.full((EPAD - EE,), NN, jnp.int32)])
    qsrc, qdst = _sc_partition()(src1d, dst1d)

    batch3d = jnp.concatenate(
        [batch.astype(jnp.int32), jnp.full((NPAD - NN,), NG, jnp.int32)]
    ).reshape(GRID, 1, ROWS)

    h = jnp.pad(x, ((0, NPAD - NN), (0, 0)))

    pooled = None
    for li, (wa, ba, wb, bb, gamma, beta) in enumerate(layers):
        Hin = h.shape[1]
        zeros_hbm = jnp.zeros((QACC // NS, Hin), jnp.float32)
        agg = _sc_agg(Hin)(qsrc, qdst, zeros_hbm, h)
        t, st = _mlp_stats(
            Hin, h, agg, wa, ba.reshape(1, 128), wb, bb.reshape(1, 128)
        )
        mean = st[0] / NN
        var = st[1] / NN - mean * mean
        inv = gamma * lax.rsqrt(var + 1e-5)
        scale = inv.reshape(1, 128)
        shift = (beta - mean * inv).reshape(1, 128)
        if li < 2:
            h = _bn_relu(t, scale, shift)
        else:
            pooled = _bn_relu_pool(t, scale, shift, batch3d)

    wc1, bc1, wc2, bc2 = head
    return _head(pooled, wc1, bc1.reshape(1, 64), wc2, bc2.reshape(1, 2))


# speedup vs baseline: 3.1205x; 3.1205x over previous
import functools

import jax
import jax.numpy as jnp
from jax import lax
from jax.experimental import pallas as pl
from jax.experimental.pallas import tpu as pltpu
from jax.experimental.pallas import tpu_sc as plsc

NN = 50000
NPAD = 50176
EE = 800000
EB = 128
NBLK = 6400
EPAD = NBLK * EB
ROWS = 1024
GRID = NPAD // ROWS
NC, NS = 2, 16
BPT = NBLK // NS
RPT = NPAD // NS
ZR = 98
NG = 64
CHK = 16
NCHK = BPT // CHK
DEPTH = 4


def _sc_agg(C):
    cpc = C // NC

    mesh = plsc.VectorSubcoreMesh(
        core_axis_name="c", subcore_axis_name="s", num_cores=NC, num_subcores=NS
    )

    @functools.partial(
        pl.kernel,
        out_type=[jax.ShapeDtypeStruct((NPAD, 32), jnp.float32)] * C,
        mesh=mesh,
        compiler_params=pltpu.CompilerParams(use_tc_tiling_on_sc=False),
        scratch_types=(
            [
                pltpu.VMEM((2, CHK, EB), jnp.int32),
                pltpu.VMEM((2, CHK, EB), jnp.int32),
                pltpu.VMEM((ZR, 32), jnp.float32),
                pltpu.VMEM_SHARED((NPAD, 32), jnp.float32),
            ]
            + [pltpu.VMEM((EB, 32), jnp.float32) for _ in range(DEPTH)]
            + [pltpu.SemaphoreType.DMA] * (2 + 2 * DEPTH)
        ),
    )
    def agg_kernel(src_hbm, dst_hbm, *rest):
        h_refs = rest[:C]
        out_refs = rest[C:2 * C]
        rest = rest[2 * C:]
        sbuf, dbuf, zbuf, acc = rest[:4]
        rows = rest[4:4 + DEPTH]
        sem_si, sem_di = rest[4 + DEPTH:6 + DEPTH]
        sem_g = rest[6 + DEPTH:6 + 2 * DEPTH]
        sem_s = rest[6 + 2 * DEPTH:6 + 3 * DEPTH]

        cid = lax.axis_index("c")
        sid = lax.axis_index("s")

        zero16 = jnp.zeros((16,), jnp.float32)

        def zinit(i, carry):
            zbuf[i, pl.ds(0, 16)] = zero16
            zbuf[i, pl.ds(16, 16)] = zero16
            return carry

        lax.fori_loop(0, ZR, zinit, 0)

        for c in range(C):

            @pl.when(cid == c // cpc)
            def _(c=c):
                h_ref = h_refs[c]
                for z in range(RPT // ZR):
                    pltpu.sync_copy(
                        zbuf, acc.at[pl.ds(sid * RPT + z * ZR, ZR)]
                    )
                plsc.subcore_barrier()

                def fire_gather(b, par, j):
                    return pltpu.async_copy(
                        h_ref.at[sbuf.at[par, j]], rows[b], sem_g[b]
                    )

                def wait_gather(b, par, j):
                    pltpu.make_async_copy(
                        h_ref.at[sbuf.at[par, j]], rows[b], sem_g[b]
                    ).wait()

                def fire_scatter(b, par, j):
                    return pltpu.async_copy(
                        rows[b], acc.at[dbuf.at[par, j]], sem_s[b], add=True
                    )

                def wait_scatter(b, par, j):
                    pltpu.make_async_copy(
                        rows[b], acc.at[dbuf.at[par, j]], sem_s[b]
                    ).wait()

                def fire_idx(k, slot):
                    base = sid * BPT + k * CHK
                    pltpu.async_copy(
                        src_hbm.at[pl.ds(base, CHK)], sbuf.at[slot], sem_si
                    )
                    pltpu.async_copy(
                        dst_hbm.at[pl.ds(base, CHK)], dbuf.at[slot], sem_di
                    )

                def wait_idx(slot):
                    pltpu.make_async_copy(
                        src_hbm.at[pl.ds(0, CHK)], sbuf.at[slot], sem_si
                    ).wait()
                    pltpu.make_async_copy(
                        dst_hbm.at[pl.ds(0, CHK)], dbuf.at[slot], sem_di
                    ).wait()

                base0 = sid * BPT
                pltpu.sync_copy(src_hbm.at[pl.ds(base0, CHK)], sbuf.at[0])
                pltpu.sync_copy(dst_hbm.at[pl.ds(base0, CHK)], dbuf.at[0])
                fire_idx(1, 1)

                def chunk_body(k, carry2):
                    par = k % 2

                    @pl.when(k > 0)
                    def _():
                        wait_idx(par)

                    @pl.when(jnp.logical_and(k > 0, k < NCHK - 1))
                    def _():
                        fire_idx(k + 1, (k + 1) % 2)

                    for b in range(DEPTH):
                        fire_gather(b, par, b)
                    for g in range(CHK // DEPTH):
                        for b in range(DEPTH):
                            wait_gather(b, par, g * DEPTH + b)
                            fire_scatter(b, par, g * DEPTH + b)
                        for b in range(DEPTH):
                            wait_scatter(b, par, g * DEPTH + b)
                            if g < CHK // DEPTH - 1:
                                fire_gather(b, par, (g + 1) * DEPTH + b)
                    return carry2

                lax.fori_loop(0, NCHK, chunk_body, 0)
                plsc.subcore_barrier()

                pltpu.sync_copy(
                    acc.at[pl.ds(sid * RPT, RPT)],
                    out_refs[c].at[pl.ds(sid * RPT, RPT)],
                )
                plsc.subcore_barrier()

        return None

    return agg_kernel


def _mlp_stats(C, h_list, agg_list, wa, ba, wb, bb):
    def body(*refs):
        h_refs = refs[:C]
        a_refs = refs[C:2 * C]
        wa_ref, ba_ref, wb_ref, bb_ref, t_ref, st_ref = refs[2 * C:]
        i = pl.program_id(0)

        acc = jnp.zeros((ROWS, 128), jnp.float32)
        for c in range(C):
            m_c = h_refs[c][...] + a_refs[c][...]
            acc = acc + jnp.dot(
                m_c,
                wa_ref[pl.ds(c * 32, 32), :],
                preferred_element_type=jnp.float32,
            )
        z = jnp.maximum(acc + ba_ref[0, :], 0.0)
        t = (
            jnp.dot(z, wb_ref[...], preferred_element_type=jnp.float32)
            + bb_ref[0, :]
        )
        t_ref[...] = t

        rowid = i * ROWS + lax.broadcasted_iota(jnp.int32, (ROWS, 1), 0)
        tm = jnp.where(rowid < NN, t, 0.0)
        s = jnp.sum(tm, axis=0)
        ss = jnp.sum(tm * tm, axis=0)
        st = jnp.concatenate([s[None, :], ss[None, :]], axis=0)

        @pl.when(i == 0)
        def _():
            st_ref[...] = jnp.zeros((2, 128), jnp.float32)

        st_ref[...] += st

    chunk_spec = pl.BlockSpec((ROWS, 32), lambda i: (i, 0))
    out = pl.pallas_call(
        body,
        grid=(GRID,),
        in_specs=(
            [chunk_spec] * (2 * C)
            + [
                pl.BlockSpec((C * 32, 128), lambda i: (0, 0)),
                pl.BlockSpec((1, 128), lambda i: (0, 0)),
                pl.BlockSpec((128, 128), lambda i: (0, 0)),
                pl.BlockSpec((1, 128), lambda i: (0, 0)),
            ]
        ),
        out_specs=[
            pl.BlockSpec((ROWS, 128), lambda i: (i, 0)),
            pl.BlockSpec((2, 128), lambda i: (0, 0)),
        ],
        out_shape=[
            jax.ShapeDtypeStruct((NPAD, 128), jnp.float32),
            jax.ShapeDtypeStruct((2, 128), jnp.float32),
        ],
    )(*h_list, *agg_list, wa, ba, wb, bb)
    return out


def _bn_relu_chunk(t, scale, shift):
    def body(t_ref, sc_ref, sh_ref, *out_refs):
        h = jnp.maximum(t_ref[...] * sc_ref[0, :] + sh_ref[0, :], 0.0)
        for c in range(4):
            out_refs[c][...] = h[:, c * 32:(c + 1) * 32]

    chunk_spec = pl.BlockSpec((ROWS, 32), lambda i: (i, 0))
    return pl.pallas_call(
        body,
        grid=(GRID,),
        in_specs=[
            pl.BlockSpec((ROWS, 128), lambda i: (i, 0)),
            pl.BlockSpec((1, 128), lambda i: (0, 0)),
            pl.BlockSpec((1, 128), lambda i: (0, 0)),
        ],
        out_specs=[chunk_spec] * 4,
        out_shape=[jax.ShapeDtypeStruct((NPAD, 32), jnp.float32)] * 4,
    )(t, scale, shift)


def _bn_relu_pool(t, scale, shift, batch3d):
    def body(t_ref, sc_ref, sh_ref, b_ref, pool_ref):
        i = pl.program_id(0)
        h = jnp.maximum(t_ref[...] * sc_ref[0, :] + sh_ref[0, :], 0.0)
        seg = b_ref[0, 0, :]
        onehot = (
            lax.broadcasted_iota(jnp.int32, (NG, ROWS), 0) == seg[None, :]
        ).astype(jnp.float32)
        part = jnp.dot(onehot, h, preferred_element_type=jnp.float32)

        @pl.when(i == 0)
        def _():
            pool_ref[...] = jnp.zeros((NG, 128), jnp.float32)

        pool_ref[...] += part

    return pl.pallas_call(
        body,
        grid=(GRID,),
        in_specs=[
            pl.BlockSpec((ROWS, 128), lambda i: (i, 0)),
            pl.BlockSpec((1, 128), lambda i: (0, 0)),
            pl.BlockSpec((1, 128), lambda i: (0, 0)),
            pl.BlockSpec((1, 1, ROWS), lambda i: (i, 0, 0)),
        ],
        out_specs=pl.BlockSpec((NG, 128), lambda i: (0, 0)),
        out_shape=jax.ShapeDtypeStruct((NG, 128), jnp.float32),
    )(t, scale, shift, batch3d)


def _head(pooled, wc1, bc1, wc2, bc2):
    def body(p_ref, w1_ref, b1_ref, w2_ref, b2_ref, o_ref):
        z1 = jnp.maximum(
            jnp.dot(p_ref[...], w1_ref[...], preferred_element_type=jnp.float32)
            + b1_ref[0, :],
            0.0,
        )
        z = (
            jnp.dot(z1, w2_ref[...], preferred_element_type=jnp.float32)
            + b2_ref[0, :]
        )
        zmax = jnp.max(z, axis=1, keepdims=True)
        lse = zmax + jnp.log(jnp.sum(jnp.exp(z - zmax), axis=1, keepdims=True))
        o_ref[...] = z - lse

    return pl.pallas_call(
        body,
        out_shape=jax.ShapeDtypeStruct((NG, 2), jnp.float32),
    )(pooled, wc1, bc1, wc2, bc2)


def kernel(x, edge_index, batch, params):
    layers, head = params
    src = edge_index[0].astype(jnp.int32)
    dst = edge_index[1].astype(jnp.int32)

    src2d = jnp.concatenate(
        [src, jnp.zeros((EPAD - EE,), jnp.int32)]
    ).reshape(NBLK, EB)
    dst2d = jnp.concatenate(
        [dst, jnp.full((EPAD - EE,), NN, jnp.int32)]
    ).reshape(NBLK, EB)

    batch3d = jnp.concatenate(
        [batch.astype(jnp.int32), jnp.full((NPAD - NN,), NG, jnp.int32)]
    ).reshape(GRID, 1, ROWS)

    h_list = [
        jnp.pad(x[:, 32 * c:32 * (c + 1)], ((0, NPAD - NN), (0, 0)))
        for c in range(2)
    ]

    pooled = None
    for li, (wa, ba, wb, bb, gamma, beta) in enumerate(layers):
        C = len(h_list)
        agg_list = _sc_agg(C)(src2d, dst2d, *h_list)
        t, st = _mlp_stats(
            C,
            h_list,
            agg_list,
            wa,
            ba.reshape(1, 128),
            wb,
            bb.reshape(1, 128),
        )
        mean = st[0] / NN
        var = st[1] / NN - mean * mean
        inv = gamma * lax.rsqrt(var + 1e-5)
        scale = inv.reshape(1, 128)
        shift = (beta - mean * inv).reshape(1, 128)
        if li < 2:
            h_list = _bn_relu_chunk(t, scale, shift)
        else:
            pooled = _bn_relu_pool(t, scale, shift, batch3d)

    wc1, bc1, wc2, bc2 = head
    return _head(
        pooled, wc1, bc1.reshape(1, 64), wc2, bc2.reshape(1, 2)
    )
